# baseline (speedup 1.0000x reference)
import jax
import jax.numpy as jnp
from jax import lax
from jax.experimental import pallas as pl
from jax.experimental.pallas import tpu as pltpu
from jax.experimental.pallas import tpu_sc as plsc

D_MODEL = 64
BLK = 128
NUM_WORKERS = 32


def _emb_body(xg_hbm, tok_hbm, pos_hbm, out_hbm,
              idx_v, pos_v, g0, g1, w0, w1, gs0, gs1, os0, os1):
    wid = lax.axis_index("s") * 2 + lax.axis_index("c")
    nchunk = xg_hbm.shape[1]
    ngrp = out_hbm.shape[2]
    niter = nchunk // 2

    pltpu.sync_copy(xg_hbm.at[wid], idx_v)
    pltpu.sync_copy(pos_hbm, pos_v)

    iota = lax.iota(jnp.int32, 16)
    dv = [(iota + 16 * j) * BLK for j in range(4)]

    def wait_gather(buf, sem):
        pltpu.make_async_copy(tok_hbm.at[pl.ds(0, BLK)], buf, sem).wait()

    def wait_out(buf, sem):
        for k in range(D_MODEL // 8):
            pltpu.make_async_copy(
                buf.at[pl.ds(1024 * k, 1024)],
                out_hbm.at[0, k, 0, :], sem).wait()

    def transpose_add(t, gbuf, wbuf):
        c = wid * nchunk + t
        s = c // ngrp
        pos_regs = [pos_v[s, pl.ds(16 * j, 16)] for j in range(4)]

        def row2(b, carry):
            for bb in (b, b + 1):
                for j in range(4):
                    v = gbuf[bb, pl.ds(16 * j, 16)] + pos_regs[j]
                    plsc.store_scatter(wbuf, [dv[j] + bb], v)
            return carry

        lax.fori_loop(0, BLK // 2, lambda i, cc: row2(2 * i, cc), 0)
        return s, c % ngrp

    def issue_gather(t, buf, sem):
        pltpu.async_copy(tok_hbm.at[idx_v.at[t]], buf, sem)

    def issue_out(s, g, wbuf, sem):
        for k in range(D_MODEL // 8):
            pltpu.async_copy(
                wbuf.at[pl.ds(1024 * k, 1024)],
                out_hbm.at[s, k, g, :], sem)

    issue_gather(0, g0, gs0)

    def body(i, carry):
        t0 = 2 * i
        t1 = t0 + 1
        wait_gather(g0, gs0)
        issue_gather(t1, g1, gs1)
        @pl.when(i > 0)
        def _():
            wait_out(w0, os0)
        s, g = transpose_add(t0, g0, w0)
        issue_out(s, g, w0, os0)
        wait_gather(g1, gs1)
        @pl.when(i < niter - 1)
        def _():
            issue_gather(t0 + 2, g0, gs0)
        @pl.when(i > 0)
        def _():
            wait_out(w1, os1)
        s, g = transpose_add(t1, g1, w1)
        issue_out(s, g, w1, os1)
        return carry

    lax.fori_loop(0, niter, body, 0)
    wait_out(w0, os0)
    wait_out(w1, os1)


def kernel(x, token_embed, pos_embed):
    batch, seq_len = x.shape
    nchunk = batch * seq_len // (NUM_WORKERS * BLK)
    ngrp = batch // BLK
    xg = x.T.reshape(NUM_WORKERS, nchunk, BLK).astype(jnp.int32)
    mesh = plsc.VectorSubcoreMesh(core_axis_name="c", subcore_axis_name="s")
    out5 = pl.kernel(
        _emb_body,
        out_type=jax.ShapeDtypeStruct(
            (seq_len, D_MODEL // 8, ngrp, 8 * BLK), jnp.float32),
        mesh=mesh,
        compiler_params=pltpu.CompilerParams(
            use_tc_tiling_on_sc=False, needs_layout_passes=False),
        scratch_types=[
            pltpu.VMEM((nchunk, BLK), jnp.int32),
            pltpu.VMEM((seq_len, D_MODEL), jnp.float32),
            pltpu.VMEM((BLK, D_MODEL), jnp.float32),
            pltpu.VMEM((BLK, D_MODEL), jnp.float32),
            pltpu.VMEM((D_MODEL * BLK,), jnp.float32),
            pltpu.VMEM((D_MODEL * BLK,), jnp.float32),
            pltpu.SemaphoreType.DMA,
            pltpu.SemaphoreType.DMA,
            pltpu.SemaphoreType.DMA,
            pltpu.SemaphoreType.DMA,
        ],
    )(xg, token_embed, pos_embed[:seq_len])
    out5 = out5.reshape(seq_len, D_MODEL // 8, ngrp, 8, BLK)
    return out5.transpose(2, 4, 0, 1, 3).reshape(batch, seq_len, D_MODEL)


# speedup vs baseline: 1.4061x; 1.4061x over previous
import jax
import jax.numpy as jnp
from jax import lax
from jax.experimental import pallas as pl
from jax.experimental.pallas import tpu as pltpu
from jax.experimental.pallas import tpu_sc as plsc

D_MODEL = 64
BLK = 128
TBLK = 128
NUM_WORKERS = 32


def _wid():
    return lax.axis_index("s") * 2 + lax.axis_index("c")


def _tr_body(tokT_hbm, tail_hbm, out_hbm, g0, g1, w0, w1, gtail,
             gs0, gs1, os0, os1):
    wid = _wid()
    vocab = tokT_hbm.shape[1]
    nblk = vocab // TBLK
    nt = (nblk + NUM_WORKERS - 1) // NUM_WORKERS
    niter = (nt + 1) // 2
    cmax = nblk - 1

    iota = lax.iota(jnp.int32, 16)
    diag = [(iota + j) & 15 for j in range(16)]

    def chunk_of(t):
        return jnp.minimum(t * NUM_WORKERS + wid, cmax)

    def transpose(gbuf, wbuf):
        for db in range(4):
            dd = [db * 16 + diag[j] for j in range(16)]

            def vb_body(vb, carry):
                vbi = vb * 16 + iota
                vbi64 = vbi * 64
                for j in range(16):
                    v = plsc.load_gather(gbuf, [dd[j], vbi])
                    plsc.store_scatter(wbuf, [vbi64 + dd[j]], v)
                return carry

            lax.fori_loop(0, TBLK // 16, vb_body, 0)

    def issue_gather(t, buf, sem):
        c = chunk_of(t)
        pltpu.async_copy(tokT_hbm.at[:, pl.ds(c * TBLK, TBLK)], buf, sem)

    def wait_gather(buf, sem):
        pltpu.make_async_copy(tokT_hbm.at[:, pl.ds(0, TBLK)], buf, sem).wait()

    def issue_out(t, wbuf, sem):
        c = chunk_of(t)
        pltpu.async_copy(wbuf, out_hbm.at[pl.ds(c * TBLK * D_MODEL, TBLK * D_MODEL)], sem)

    def wait_out(wbuf, sem):
        pltpu.make_async_copy(wbuf, out_hbm.at[pl.ds(0, TBLK * D_MODEL)], sem).wait()

    @pl.when(wid == 0)
    def _():
        pltpu.sync_copy(tail_hbm, gtail)

        def tail_row(v, carry):
            for m in range(4):
                w0[pl.ds(v * D_MODEL + 16 * m, 16)] = gtail[v, pl.ds(16 * m, 16)]
            return carry

        lax.fori_loop(0, gtail.shape[0], tail_row, 0)
        ntail = gtail.shape[0] * D_MODEL
        pltpu.sync_copy(
            w0.at[pl.ds(0, ntail)],
            out_hbm.at[pl.ds(nblk * TBLK * D_MODEL, ntail)])

    issue_gather(0, g0, gs0)

    def body(i, carry):
        t0 = 2 * i
        t1 = t0 + 1
        wait_gather(g0, gs0)
        issue_gather(t1, g1, gs1)
        @pl.when(i > 0)
        def _():
            wait_out(w0, os0)
        transpose(g0, w0)
        issue_out(t0, w0, os0)
        wait_gather(g1, gs1)
        @pl.when(i < niter - 1)
        def _():
            issue_gather(t0 + 2, g0, gs0)
        @pl.when(i > 0)
        def _():
            wait_out(w1, os1)
        transpose(g1, w1)
        issue_out(t1, w1, os1)
        return carry

    lax.fori_loop(0, niter, body, 0)
    wait_out(w0, os0)
    wait_out(w1, os1)


def _emb_body(xg_hbm, tok_hbm, pos_hbm, out_hbm,
              idx_v, pos_v, g0, g1, w0, w1, gs0, gs1, os0, os1):
    wid = _wid()
    nchunk = xg_hbm.shape[1]
    ngrp = out_hbm.shape[2]
    niter = nchunk // 2

    pltpu.sync_copy(xg_hbm.at[wid], idx_v)
    pltpu.sync_copy(pos_hbm, pos_v)

    iota = lax.iota(jnp.int32, 16)
    diag = [(iota + j) & 15 for j in range(16)]

    def transpose_add(t, gbuf, wbuf):
        c = wid * nchunk + t
        s = c // ngrp
        sbase = s * D_MODEL
        for db in range(4):
            dd = [db * 16 + diag[j] for j in range(16)]
            pv = [plsc.load_gather(pos_v, [sbase + dd[j]]) for j in range(16)]
            wd = [dd[j] * BLK for j in range(16)]

            def bb_body(bb, carry):
                bbi = bb * 16 + iota
                for j in range(16):
                    v = plsc.load_gather(gbuf, [bbi, dd[j]]) + pv[j]
                    plsc.store_scatter(wbuf, [wd[j] + bbi], v)
                return carry

            lax.fori_loop(0, BLK // 16, bb_body, 0)
        return s, c % ngrp

    def issue_gather(t, buf, sem):
        pltpu.async_copy(tok_hbm.at[idx_v.at[t]], buf, sem)

    def wait_gather(buf, sem):
        pltpu.make_async_copy(tok_hbm.at[pl.ds(0, BLK)], buf, sem).wait()

    def issue_out(s, g, wbuf, sem):
        for k in range(D_MODEL // 8):
            pltpu.async_copy(
                wbuf.at[pl.ds(1024 * k, 1024)], out_hbm.at[s, k, g, :], sem)

    def wait_out(wbuf, sem):
        for k in range(D_MODEL // 8):
            pltpu.make_async_copy(
                wbuf.at[pl.ds(1024 * k, 1024)], out_hbm.at[0, k, 0, :], sem).wait()

    issue_gather(0, g0, gs0)

    def body(i, carry):
        t0 = 2 * i
        t1 = t0 + 1
        wait_gather(g0, gs0)
        issue_gather(t1, g1, gs1)
        @pl.when(i > 0)
        def _():
            wait_out(w0, os0)
        s, g = transpose_add(t0, g0, w0)
        issue_out(s, g, w0, os0)
        wait_gather(g1, gs1)
        @pl.when(i < niter - 1)
        def _():
            issue_gather(t0 + 2, g0, gs0)
        @pl.when(i > 0)
        def _():
            wait_out(w1, os1)
        s, g = transpose_add(t1, g1, w1)
        issue_out(s, g, w1, os1)
        return carry

    lax.fori_loop(0, niter, body, 0)
    wait_out(w0, os0)
    wait_out(w1, os1)


def kernel(x, token_embed, pos_embed):
    batch, seq_len = x.shape
    vocab = token_embed.shape[0]
    nchunk = batch * seq_len // (NUM_WORKERS * BLK)
    ngrp = batch // BLK
    mesh = plsc.VectorSubcoreMesh(core_axis_name="c", subcore_axis_name="s")

    flat = pl.kernel(
        _tr_body,
        out_type=jax.ShapeDtypeStruct((vocab * D_MODEL,), jnp.float32),
        mesh=mesh,
        compiler_params=pltpu.CompilerParams(
            use_tc_tiling_on_sc=True, needs_layout_passes=False),
        scratch_types=[
            pltpu.VMEM((D_MODEL, TBLK), jnp.float32),
            pltpu.VMEM((D_MODEL, TBLK), jnp.float32),
            pltpu.VMEM((TBLK * D_MODEL,), jnp.float32),
            pltpu.VMEM((TBLK * D_MODEL,), jnp.float32),
            pltpu.VMEM((vocab % TBLK, D_MODEL), jnp.float32),
            pltpu.SemaphoreType.DMA,
            pltpu.SemaphoreType.DMA,
            pltpu.SemaphoreType.DMA,
            pltpu.SemaphoreType.DMA,
        ],
    )(token_embed.T, token_embed[vocab - vocab % TBLK:])
    tok_rm = flat.reshape(vocab, D_MODEL)

    xg = x.T.reshape(NUM_WORKERS, nchunk, BLK).astype(jnp.int32)
    out5 = pl.kernel(
        _emb_body,
        out_type=jax.ShapeDtypeStruct(
            (seq_len, D_MODEL // 8, ngrp, 8 * BLK), jnp.float32),
        mesh=mesh,
        compiler_params=pltpu.CompilerParams(
            use_tc_tiling_on_sc=False, needs_layout_passes=False),
        scratch_types=[
            pltpu.VMEM((nchunk, BLK), jnp.int32),
            pltpu.VMEM((seq_len * D_MODEL,), jnp.float32),
            pltpu.VMEM((BLK, D_MODEL), jnp.float32),
            pltpu.VMEM((BLK, D_MODEL), jnp.float32),
            pltpu.VMEM((D_MODEL * BLK,), jnp.float32),
            pltpu.VMEM((D_MODEL * BLK,), jnp.float32),
            pltpu.SemaphoreType.DMA,
            pltpu.SemaphoreType.DMA,
            pltpu.SemaphoreType.DMA,
            pltpu.SemaphoreType.DMA,
        ],
    )(xg, tok_rm, pos_embed[:seq_len].reshape(-1))
    out5 = out5.reshape(seq_len, D_MODEL // 8, ngrp, 8, BLK)
    return out5.transpose(2, 4, 0, 1, 3).reshape(batch, seq_len, D_MODEL)


# speedup vs baseline: 2.8591x; 2.0333x over previous
import jax
import jax.numpy as jnp
from jax import lax
from jax.experimental import pallas as pl
from jax.experimental.pallas import tpu as pltpu
from jax.experimental.pallas import tpu_sc as plsc

D_MODEL = 64
BLK = 128
TBLK = 128
NUM_WORKERS = 32


def _wid():
    return lax.axis_index("s") * 2 + lax.axis_index("c")


def _tr_body(tokT_hbm, tail_hbm, out_hbm, g0, g1, w0, w1, gtail,
             gs0, gs1, os0, os1):
    wid = _wid()
    vocab = tokT_hbm.shape[1]
    nblk = vocab // TBLK
    nt = (nblk + NUM_WORKERS - 1) // NUM_WORKERS
    niter = (nt + 1) // 2
    cmax = nblk - 1

    iota = lax.iota(jnp.int32, 16)
    diag = [(iota + j) & 15 for j in range(16)]

    def chunk_of(t):
        return jnp.minimum(t * NUM_WORKERS + wid, cmax)

    def transpose(gbuf, wbuf):
        for db in range(4):
            dd = [db * 16 + diag[j] for j in range(16)]

            @plsc.parallel_loop(0, TBLK // 16, unroll=4)
            def vb_body(vb):
                vbi = vb * 16 + iota
                vbi64 = vbi * 64
                for j in range(16):
                    v = plsc.load_gather(gbuf, [dd[j], vbi])
                    plsc.store_scatter(wbuf, [vbi64 + dd[j]], v)

    def issue_gather(t, buf, sem):
        c = chunk_of(t)
        pltpu.async_copy(tokT_hbm.at[:, pl.ds(c * TBLK, TBLK)], buf, sem)

    def wait_gather(buf, sem):
        pltpu.make_async_copy(tokT_hbm.at[:, pl.ds(0, TBLK)], buf, sem).wait()

    def issue_out(t, wbuf, sem):
        c = chunk_of(t)
        pltpu.async_copy(wbuf, out_hbm.at[pl.ds(c * TBLK * D_MODEL, TBLK * D_MODEL)], sem)

    def wait_out(wbuf, sem):
        pltpu.make_async_copy(wbuf, out_hbm.at[pl.ds(0, TBLK * D_MODEL)], sem).wait()

    @pl.when(wid == 0)
    def _():
        pltpu.sync_copy(tail_hbm, gtail)

        def tail_row(v, carry):
            for m in range(4):
                w0[pl.ds(v * D_MODEL + 16 * m, 16)] = gtail[v, pl.ds(16 * m, 16)]
            return carry

        lax.fori_loop(0, gtail.shape[0], tail_row, 0)
        ntail = gtail.shape[0] * D_MODEL
        pltpu.sync_copy(
            w0.at[pl.ds(0, ntail)],
            out_hbm.at[pl.ds(nblk * TBLK * D_MODEL, ntail)])

    issue_gather(0, g0, gs0)

    def body(i, carry):
        t0 = 2 * i
        t1 = t0 + 1
        wait_gather(g0, gs0)
        issue_gather(t1, g1, gs1)
        @pl.when(i > 0)
        def _():
            wait_out(w0, os0)
        transpose(g0, w0)
        issue_out(t0, w0, os0)
        wait_gather(g1, gs1)
        @pl.when(i < niter - 1)
        def _():
            issue_gather(t0 + 2, g0, gs0)
        @pl.when(i > 0)
        def _():
            wait_out(w1, os1)
        transpose(g1, w1)
        issue_out(t1, w1, os1)
        return carry

    lax.fori_loop(0, niter, body, 0)
    wait_out(w0, os0)
    wait_out(w1, os1)


def _emb_body(xg_hbm, tok_hbm, pos_hbm, out_hbm,
              idx_v, pos_v, g0, g1, w0, w1, gs0, gs1, os0, os1):
    wid = _wid()
    nchunk = xg_hbm.shape[1]
    ngrp = out_hbm.shape[2]
    niter = nchunk // 2

    pltpu.sync_copy(xg_hbm.at[wid], idx_v)
    pltpu.sync_copy(pos_hbm, pos_v)

    iota = lax.iota(jnp.int32, 16)
    diag = [(iota + j) & 15 for j in range(16)]

    def transpose_add(t, gbuf, wbuf):
        c = wid * nchunk + t
        s = c // ngrp
        sbase = s * D_MODEL
        for db in range(4):
            dd = [db * 16 + diag[j] for j in range(16)]
            pv = [plsc.load_gather(pos_v, [sbase + dd[j]]) for j in range(16)]
            wd = [dd[j] * BLK for j in range(16)]

            @plsc.parallel_loop(0, BLK // 16, unroll=4)
            def bb_body(bb):
                bbi = bb * 16 + iota
                for j in range(16):
                    v = plsc.load_gather(gbuf, [bbi, dd[j]]) + pv[j]
                    plsc.store_scatter(wbuf, [wd[j] + bbi], v)
        return s, c % ngrp

    def issue_gather(t, buf, sem):
        pltpu.async_copy(tok_hbm.at[idx_v.at[t]], buf, sem)

    def wait_gather(buf, sem):
        pltpu.make_async_copy(tok_hbm.at[pl.ds(0, BLK)], buf, sem).wait()

    def issue_out(s, g, wbuf, sem):
        for k in range(D_MODEL // 8):
            pltpu.async_copy(
                wbuf.at[pl.ds(1024 * k, 1024)], out_hbm.at[s, k, g, :], sem)

    def wait_out(wbuf, sem):
        for k in range(D_MODEL // 8):
            pltpu.make_async_copy(
                wbuf.at[pl.ds(1024 * k, 1024)], out_hbm.at[0, k, 0, :], sem).wait()

    issue_gather(0, g0, gs0)

    def body(i, carry):
        t0 = 2 * i
        t1 = t0 + 1
        wait_gather(g0, gs0)
        issue_gather(t1, g1, gs1)
        @pl.when(i > 0)
        def _():
            wait_out(w0, os0)
        s, g = transpose_add(t0, g0, w0)
        issue_out(s, g, w0, os0)
        wait_gather(g1, gs1)
        @pl.when(i < niter - 1)
        def _():
            issue_gather(t0 + 2, g0, gs0)
        @pl.when(i > 0)
        def _():
            wait_out(w1, os1)
        s, g = transpose_add(t1, g1, w1)
        issue_out(s, g, w1, os1)
        return carry

    lax.fori_loop(0, niter, body, 0)
    wait_out(w0, os0)
    wait_out(w1, os1)


def kernel(x, token_embed, pos_embed):
    batch, seq_len = x.shape
    vocab = token_embed.shape[0]
    nchunk = batch * seq_len // (NUM_WORKERS * BLK)
    ngrp = batch // BLK
    mesh = plsc.VectorSubcoreMesh(core_axis_name="c", subcore_axis_name="s")

    flat = pl.kernel(
        _tr_body,
        out_type=jax.ShapeDtypeStruct((vocab * D_MODEL,), jnp.float32),
        mesh=mesh,
        compiler_params=pltpu.CompilerParams(
            use_tc_tiling_on_sc=True, needs_layout_passes=False),
        scratch_types=[
            pltpu.VMEM((D_MODEL, TBLK), jnp.float32),
            pltpu.VMEM((D_MODEL, TBLK), jnp.float32),
            pltpu.VMEM((TBLK * D_MODEL,), jnp.float32),
            pltpu.VMEM((TBLK * D_MODEL,), jnp.float32),
            pltpu.VMEM((vocab % TBLK, D_MODEL), jnp.float32),
            pltpu.SemaphoreType.DMA,
            pltpu.SemaphoreType.DMA,
            pltpu.SemaphoreType.DMA,
            pltpu.SemaphoreType.DMA,
        ],
    )(token_embed.T, token_embed[vocab - vocab % TBLK:])
    tok_rm = flat.reshape(vocab, D_MODEL)

    xg = x.T.reshape(NUM_WORKERS, nchunk, BLK).astype(jnp.int32)
    out5 = pl.kernel(
        _emb_body,
        out_type=jax.ShapeDtypeStruct(
            (seq_len, D_MODEL // 8, ngrp, 8 * BLK), jnp.float32),
        mesh=mesh,
        compiler_params=pltpu.CompilerParams(
            use_tc_tiling_on_sc=False, needs_layout_passes=False),
        scratch_types=[
            pltpu.VMEM((nchunk, BLK), jnp.int32),
            pltpu.VMEM((seq_len * D_MODEL,), jnp.float32),
            pltpu.VMEM((BLK, D_MODEL), jnp.float32),
            pltpu.VMEM((BLK, D_MODEL), jnp.float32),
            pltpu.VMEM((D_MODEL * BLK,), jnp.float32),
            pltpu.VMEM((D_MODEL * BLK,), jnp.float32),
            pltpu.SemaphoreType.DMA,
            pltpu.SemaphoreType.DMA,
            pltpu.SemaphoreType.DMA,
            pltpu.SemaphoreType.DMA,
        ],
    )(xg, tok_rm, pos_embed[:seq_len].reshape(-1))
    out5 = out5.reshape(seq_len, D_MODEL // 8, ngrp, 8, BLK)
    return out5.transpose(2, 4, 0, 1, 3).reshape(batch, seq_len, D_MODEL)
